# baseline (speedup 1.0000x reference)
import functools

import jax
import jax.numpy as jnp
from jax import lax
from jax.experimental import pallas as pl
from jax.experimental.pallas import tpu as pltpu
from jax.experimental.pallas import tpu_sc as plsc

BATCH = 16384
D = 16
NC = 2
NS = 16
NW = NC * NS
BW = BATCH // NW
CHUNK = 128
NCHUNK = BW // CHUNK

_mesh = plsc.VectorSubcoreMesh(core_axis_name="c", subcore_axis_name="s")


@functools.partial(
    pl.kernel,
    mesh=_mesh,
    out_type=jax.ShapeDtypeStruct((BATCH,), jnp.float32),
    scratch_types=[
        pltpu.VMEM((NCHUNK, CHUNK), jnp.int32),
        pltpu.VMEM((NCHUNK, CHUNK), jnp.int32),
        pltpu.VMEM((D, BW), jnp.float32),
        pltpu.VMEM((D, BW), jnp.float32),
        pltpu.VMEM((BW,), jnp.float32),
        pltpu.SemaphoreType.DMA,
        pltpu.SemaphoreType.DMA,
    ],
    compiler_params=pltpu.CompilerParams(
        needs_layout_passes=False, use_tc_tiling_on_sc=False),
)
def _mf_sc(xu_hbm, xv_hbm, uT_hbm, vT_hbm, out_hbm,
           xu_v, xv_v, ucols, vcols, out_v, sem_idx, sem):
    wid = lax.axis_index("s") * NC + lax.axis_index("c")
    base = wid * BW

    cpu = pltpu.async_copy(xu_hbm.at[pl.ds(wid * NCHUNK, NCHUNK), :], xu_v,
                           sem_idx)
    cpv = pltpu.async_copy(xv_hbm.at[pl.ds(wid * NCHUNK, NCHUNK), :], xv_v,
                           sem_idx)
    cpu.wait()
    cpv.wait()

    for p in range(D):
        for j in range(NCHUNK):
            s = pl.ds(j * CHUNK, CHUNK)
            pltpu.async_copy(uT_hbm.at[p].at[xu_v.at[j]], ucols.at[p, s], sem)
            pltpu.async_copy(vT_hbm.at[p].at[xv_v.at[j]], vcols.at[p, s], sem)

    for _ in range(2 * D):
        pltpu.make_async_copy(uT_hbm.at[0].at[pl.ds(0, BW)], ucols.at[0],
                              sem).wait()

    def dot_blk(g, carry):
        s = pl.ds(g * 16, 16)
        acc = ucols[0, s] * vcols[0, s]
        for p in range(1, D):
            acc = acc + ucols[p, s] * vcols[p, s]
        out_v[s] = acc
        return carry
    lax.fori_loop(0, BW // 16, dot_blk, 0, unroll=2)

    pltpu.sync_copy(out_v, out_hbm.at[pl.ds(base, BW)])


def kernel(X, user_emb, item_emb):
    xu = X[:, 0].reshape(NW * NCHUNK, CHUNK)
    xv = X[:, 1].reshape(NW * NCHUNK, CHUNK)
    out = _mf_sc(xu, xv, user_emb.T, item_emb.T)
    return out.reshape(BATCH, 1)


# speedup vs baseline: 18.9955x; 18.9955x over previous
import functools

import jax
import jax.numpy as jnp
from jax import lax
from jax.experimental import pallas as pl
from jax.experimental.pallas import tpu as pltpu
from jax.experimental.pallas import tpu_sc as plsc

BATCH = 16384
D = 16
NC = 2
NS = 16
NW = NC * NS
BW = BATCH // NW
G = 8
NG = BW // G

_mesh = plsc.VectorSubcoreMesh(core_axis_name="c", subcore_axis_name="s")


@functools.partial(
    pl.kernel,
    mesh=_mesh,
    out_type=jax.ShapeDtypeStruct((BATCH,), jnp.float32),
    scratch_types=[
        pltpu.VMEM((BW + 16,), jnp.int32),
        pltpu.VMEM((BW + 16,), jnp.int32),
        pltpu.VMEM((2, G, D, 128), jnp.float32),
        pltpu.VMEM((2, G, D, 128), jnp.float32),
        pltpu.VMEM((16, D), jnp.float32),
        pltpu.VMEM((BW,), jnp.float32),
        pltpu.SemaphoreType.DMA,
        pltpu.SemaphoreType.DMA,
        pltpu.SemaphoreType.DMA,
    ],
    compiler_params=pltpu.CompilerParams(needs_layout_passes=False),
)
def _mf_sc(xu_hbm, xv_hbm, uT_hbm, vT_hbm, out_hbm,
           xu_f, xv_f, ubuf, vbuf, prods, out_v,
           sem_idx, sem_a, sem_b):
    wid = lax.axis_index("s") * NC + lax.axis_index("c")
    base = pl.multiple_of(wid * BW, 128)

    pltpu.async_copy(xu_hbm.at[pl.ds(base, BW)], xu_f.at[pl.ds(0, BW)],
                     sem_idx).wait()
    pltpu.async_copy(xv_hbm.at[pl.ds(base, BW)], xv_f.at[pl.ds(0, BW)],
                     sem_idx).wait()

    lane_ids = lax.iota(jnp.int32, 16)

    def fire(g, sem):
        buf = g & 1
        xu16 = xu_f[pl.ds(g * G, 16)]
        xv16 = xv_f[pl.ds(g * G, 16)]
        for k in range(G):
            cu0 = pl.multiple_of((xu16[k] >> 7) << 7, 128)
            cv0 = pl.multiple_of((xv16[k] >> 7) << 7, 128)
            pltpu.async_copy(uT_hbm.at[:, pl.ds(cu0, 128)],
                             ubuf.at[buf, k], sem)
            pltpu.async_copy(vT_hbm.at[:, pl.ds(cv0, 128)],
                             vbuf.at[buf, k], sem)

    def drain(sem):
        for _ in range(2 * G):
            pltpu.make_async_copy(uT_hbm.at[:, pl.ds(0, 128)],
                                  ubuf.at[0, 0], sem).wait()

    def extract(g):
        buf = g & 1
        xu16 = xu_f[pl.ds(g * G, 16)]
        xv16 = xv_f[pl.ds(g * G, 16)]
        prow = buf * G
        for k in range(G):
            bu = jnp.full((16,), buf, jnp.int32)
            ku = jnp.full((16,), k, jnp.int32)
            cu = jnp.full((16,), xu16[k] & 127, jnp.int32)
            cv = jnp.full((16,), xv16[k] & 127, jnp.int32)
            fu = plsc.load_gather(ubuf, [bu, ku, lane_ids, cu])
            fv = plsc.load_gather(vbuf, [bu, ku, lane_ids, cv])
            prods[prow + k] = fu * fv

    def reduce16(g):
        acc = jnp.zeros((16,), jnp.float32)
        for d in range(D):
            cols = jnp.full((16,), d, jnp.int32)
            acc = acc + plsc.load_gather(prods, [lane_ids, cols])
        out_v[pl.ds((g - 2) * G, 16)] = acc

    fire(0, sem_a)

    def body(g, carry):
        @pl.when(g < NG)
        def _():
            @pl.when((g & 1) == 1)
            def _():
                fire(g, sem_b)

            @pl.when((g & 1) == 0)
            def _():
                fire(g, sem_a)

        @pl.when(((g - 1) & 1) == 1)
        def _():
            drain(sem_b)

        @pl.when(((g - 1) & 1) == 0)
        def _():
            drain(sem_a)
        extract(g - 1)

        @pl.when((g & 1) == 0)
        def _():
            reduce16(g)
        return carry

    lax.fori_loop(1, NG + 1, body, 0)

    pltpu.sync_copy(out_v, out_hbm.at[pl.ds(base, BW)])


def kernel(X, user_emb, item_emb):
    xu = X[:, 0]
    xv = X[:, 1]
    out = _mf_sc(xu, xv, user_emb.T, item_emb.T)
    return out.reshape(BATCH, 1)


# speedup vs baseline: 20.6622x; 1.0877x over previous
import functools

import jax
import jax.numpy as jnp
from jax import lax
from jax.experimental import pallas as pl
from jax.experimental.pallas import tpu as pltpu
from jax.experimental.pallas import tpu_sc as plsc

BATCH = 16384
D = 16
NC = 2
NS = 16
NW = NC * NS
BW = BATCH // NW
G = 8
NG = BW // G
NB = 3

_mesh = plsc.VectorSubcoreMesh(core_axis_name="c", subcore_axis_name="s")


@functools.partial(
    pl.kernel,
    mesh=_mesh,
    out_type=jax.ShapeDtypeStruct((BATCH,), jnp.float32),
    scratch_types=[
        pltpu.VMEM((BW + 16,), jnp.int32),
        pltpu.VMEM((BW + 16,), jnp.int32),
        pltpu.VMEM((NB, G, D, 128), jnp.float32),
        pltpu.VMEM((NB, G, D, 128), jnp.float32),
        pltpu.VMEM((16, D), jnp.float32),
        pltpu.VMEM((BW,), jnp.float32),
        pltpu.SemaphoreType.DMA,
        pltpu.SemaphoreType.DMA,
        pltpu.SemaphoreType.DMA,
        pltpu.SemaphoreType.DMA,
    ],
    compiler_params=pltpu.CompilerParams(needs_layout_passes=False),
)
def _mf_sc(xu_hbm, xv_hbm, uT_hbm, vT_hbm, out_hbm,
           xu_f, xv_f, ubuf, vbuf, prods, out_v,
           sem_idx, sem_a, sem_b, sem_c):
    wid = lax.axis_index("s") * NC + lax.axis_index("c")
    base = pl.multiple_of(wid * BW, 128)

    pltpu.async_copy(xu_hbm.at[pl.ds(base, BW)], xu_f.at[pl.ds(0, BW)],
                     sem_idx).wait()
    pltpu.async_copy(xv_hbm.at[pl.ds(base, BW)], xv_f.at[pl.ds(0, BW)],
                     sem_idx).wait()

    lane_ids = lax.iota(jnp.int32, 16)
    sems = (sem_a, sem_b, sem_c)

    def fire(g, slot):
        xu16 = xu_f[pl.ds(g * G, 16)]
        xv16 = xv_f[pl.ds(g * G, 16)]
        for k in range(G):
            cu0 = pl.multiple_of((xu16[k] >> 7) << 7, 128)
            cv0 = pl.multiple_of((xv16[k] >> 7) << 7, 128)
            pltpu.async_copy(uT_hbm.at[:, pl.ds(cu0, 128)],
                             ubuf.at[slot, k], sems[slot])
            pltpu.async_copy(vT_hbm.at[:, pl.ds(cv0, 128)],
                             vbuf.at[slot, k], sems[slot])

    def drain(slot):
        for _ in range(2 * G):
            pltpu.make_async_copy(uT_hbm.at[:, pl.ds(0, 128)],
                                  ubuf.at[0, 0], sems[slot]).wait()

    def extract(e, slot):
        xu16 = xu_f[pl.ds(e * G, 16)]
        xv16 = xv_f[pl.ds(e * G, 16)]
        prow = (e & 1) * G
        bu = jnp.full((16,), slot, jnp.int32)
        for k in range(G):
            ku = jnp.full((16,), k, jnp.int32)
            cu = jnp.full((16,), xu16[k] & 127, jnp.int32)
            cv = jnp.full((16,), xv16[k] & 127, jnp.int32)
            fu = plsc.load_gather(ubuf, [bu, ku, lane_ids, cu])
            fv = plsc.load_gather(vbuf, [bu, ku, lane_ids, cv])
            prods[prow + k] = fu * fv

    def reduce16(e):
        acc = jnp.zeros((16,), jnp.float32)
        for d in range(D):
            cols = jnp.full((16,), d, jnp.int32)
            acc = acc + plsc.load_gather(prods, [lane_ids, cols])
        out_v[pl.ds((e - 1) * G, 16)] = acc

    fire(0, 0)
    fire(1, 1)

    def body(g, carry):
        e = g - 2

        @pl.when(g < NG)
        def _():
            for r in range(NB):
                @pl.when(lax.rem(g, NB) == r)
                def _(r=r):
                    fire(g, r)

        for r in range(NB):
            @pl.when(lax.rem(e, NB) == r)
            def _(r=r):
                drain(r)
        extract(e, lax.rem(e, NB))

        @pl.when((e & 1) == 1)
        def _():
            reduce16(e)
        return carry

    lax.fori_loop(2, NG + 2, body, 0)

    pltpu.sync_copy(out_v, out_hbm.at[pl.ds(base, BW)])


def kernel(X, user_emb, item_emb):
    xu = X[:, 0]
    xv = X[:, 1]
    out = _mf_sc(xu, xv, user_emb.T, item_emb.T)
    return out.reshape(BATCH, 1)


# speedup vs baseline: 22.4940x; 1.0887x over previous
import functools

import jax
import jax.numpy as jnp
from jax import lax
from jax.experimental import pallas as pl
from jax.experimental.pallas import tpu as pltpu
from jax.experimental.pallas import tpu_sc as plsc

BATCH = 16384
D = 16
NC = 2
NS = 16
NW = NC * NS
BW = BATCH // NW
G = 4
NG = BW // G
NB = 6
NF = NB - 1
GPR = 16 // G

_mesh = plsc.VectorSubcoreMesh(core_axis_name="c", subcore_axis_name="s")


@functools.partial(
    pl.kernel,
    mesh=_mesh,
    out_type=jax.ShapeDtypeStruct((BATCH,), jnp.float32),
    scratch_types=[
        pltpu.VMEM((BW + 16,), jnp.int32),
        pltpu.VMEM((BW + 16,), jnp.int32),
        pltpu.VMEM((NB, G, D, 128), jnp.float32),
        pltpu.VMEM((NB, G, D, 128), jnp.float32),
        pltpu.VMEM((16, D), jnp.float32),
        pltpu.VMEM((BW,), jnp.float32),
        pltpu.SemaphoreType.DMA,
        pltpu.SemaphoreType.DMA,
        pltpu.SemaphoreType.DMA,
        pltpu.SemaphoreType.DMA,
        pltpu.SemaphoreType.DMA,
        pltpu.SemaphoreType.DMA,
        pltpu.SemaphoreType.DMA,
    ],
    compiler_params=pltpu.CompilerParams(needs_layout_passes=False),
)
def _mf_sc(xu_hbm, xv_hbm, uT_hbm, vT_hbm, out_hbm,
           xu_f, xv_f, ubuf, vbuf, prods, out_v,
           sem_idx, sem_a, sem_b, sem_c, sem_d, sem_e, sem_f):
    wid = lax.axis_index("s") * NC + lax.axis_index("c")
    base = pl.multiple_of(wid * BW, 128)

    pltpu.async_copy(xu_hbm.at[pl.ds(base, BW)], xu_f.at[pl.ds(0, BW)],
                     sem_idx).wait()
    pltpu.async_copy(xv_hbm.at[pl.ds(base, BW)], xv_f.at[pl.ds(0, BW)],
                     sem_idx).wait()

    lane_ids = lax.iota(jnp.int32, 16)
    sems = (sem_a, sem_b, sem_c, sem_d, sem_e, sem_f)

    def fire(g, slot):
        xu16 = xu_f[pl.ds(g * G, 16)]
        xv16 = xv_f[pl.ds(g * G, 16)]
        for k in range(G):
            cu0 = pl.multiple_of((xu16[k] >> 7) << 7, 128)
            cv0 = pl.multiple_of((xv16[k] >> 7) << 7, 128)
            pltpu.async_copy(uT_hbm.at[:, pl.ds(cu0, 128)],
                             ubuf.at[slot, k], sems[slot])
            pltpu.async_copy(vT_hbm.at[:, pl.ds(cv0, 128)],
                             vbuf.at[slot, k], sems[slot])

    def drain(slot):
        pltpu.make_async_copy(uT_hbm.at[:, pl.ds(0, 128)],
                              ubuf.at[0], sems[slot]).wait()
        pltpu.make_async_copy(uT_hbm.at[:, pl.ds(0, 128)],
                              vbuf.at[0], sems[slot]).wait()

    def extract(e, slot):
        xu16 = xu_f[pl.ds(e * G, 16)]
        xv16 = xv_f[pl.ds(e * G, 16)]
        prow = lax.rem(e, GPR) * G
        bu = jnp.full((16,), slot, jnp.int32)
        for k in range(G):
            ku = jnp.full((16,), k, jnp.int32)
            cu = jnp.full((16,), xu16[k] & 127, jnp.int32)
            cv = jnp.full((16,), xv16[k] & 127, jnp.int32)
            fu = plsc.load_gather(ubuf, [bu, ku, lane_ids, cu])
            fv = plsc.load_gather(vbuf, [bu, ku, lane_ids, cv])
            prods[prow + k] = fu * fv

    def reduce16(e):
        acc = jnp.zeros((16,), jnp.float32)
        for d in range(D):
            cols = jnp.full((16,), d, jnp.int32)
            acc = acc + plsc.load_gather(prods, [lane_ids, cols])
        out_v[pl.ds((e - (GPR - 1)) * G, 16)] = acc

    for p in range(NF):
        fire(p, p)

    def body(g, carry):
        e = g - NF

        @pl.when(g < NG)
        def _():
            for r in range(NB):
                @pl.when(lax.rem(g, NB) == r)
                def _(r=r):
                    fire(g, r)

        for r in range(NB):
            @pl.when(lax.rem(e, NB) == r)
            def _(r=r):
                drain(r)
        extract(e, lax.rem(e, NB))

        @pl.when(lax.rem(e, GPR) == GPR - 1)
        def _():
            reduce16(e)
        return carry

    lax.fori_loop(NF, NG + NF, body, 0)

    pltpu.sync_copy(out_v, out_hbm.at[pl.ds(base, BW)])


def kernel(X, user_emb, item_emb):
    xu = X[:, 0]
    xv = X[:, 1]
    out = _mf_sc(xu, xv, user_emb.T, item_emb.T)
    return out.reshape(BATCH, 1)


# speedup vs baseline: 22.6601x; 1.0074x over previous
import functools

import jax
import jax.numpy as jnp
from jax import lax
from jax.experimental import pallas as pl
from jax.experimental.pallas import tpu as pltpu
from jax.experimental.pallas import tpu_sc as plsc

BATCH = 16384
D = 16
NC = 2
NS = 16
NW = NC * NS
BW = BATCH // NW
G = 2
NG = BW // G
NB = 12
NF = NB - 1
GPR = 16 // G

_mesh = plsc.VectorSubcoreMesh(core_axis_name="c", subcore_axis_name="s")


@functools.partial(
    pl.kernel,
    mesh=_mesh,
    out_type=jax.ShapeDtypeStruct((BATCH,), jnp.float32),
    scratch_types=[
        pltpu.VMEM((BW + 16,), jnp.int32),
        pltpu.VMEM((BW + 16,), jnp.int32),
        pltpu.VMEM((NB, G, D, 128), jnp.float32),
        pltpu.VMEM((NB, G, D, 128), jnp.float32),
        pltpu.VMEM((16, D), jnp.float32),
        pltpu.VMEM((BW,), jnp.float32),
    ] + [pltpu.SemaphoreType.DMA] * (1 + NB),
    compiler_params=pltpu.CompilerParams(needs_layout_passes=False),
)
def _mf_sc(xu_hbm, xv_hbm, uT_hbm, vT_hbm, out_hbm,
           xu_f, xv_f, ubuf, vbuf, prods, out_v,
           sem_idx, *ring_sems):
    wid = lax.axis_index("s") * NC + lax.axis_index("c")
    base = pl.multiple_of(wid * BW, 128)

    pltpu.async_copy(xu_hbm.at[pl.ds(base, BW)], xu_f.at[pl.ds(0, BW)],
                     sem_idx).wait()
    pltpu.async_copy(xv_hbm.at[pl.ds(base, BW)], xv_f.at[pl.ds(0, BW)],
                     sem_idx).wait()

    lane_ids = lax.iota(jnp.int32, 16)
    sems = tuple(ring_sems)

    def fire(g, slot):
        xu16 = xu_f[pl.ds(g * G, 16)]
        xv16 = xv_f[pl.ds(g * G, 16)]
        for k in range(G):
            cu0 = pl.multiple_of((xu16[k] >> 7) << 7, 128)
            cv0 = pl.multiple_of((xv16[k] >> 7) << 7, 128)
            pltpu.async_copy(uT_hbm.at[:, pl.ds(cu0, 128)],
                             ubuf.at[slot, k], sems[slot])
            pltpu.async_copy(vT_hbm.at[:, pl.ds(cv0, 128)],
                             vbuf.at[slot, k], sems[slot])

    def drain(slot):
        pltpu.make_async_copy(uT_hbm.at[:, pl.ds(0, 128)],
                              ubuf.at[0], sems[slot]).wait()
        pltpu.make_async_copy(uT_hbm.at[:, pl.ds(0, 128)],
                              vbuf.at[0], sems[slot]).wait()

    def extract(e, slot):
        xu16 = xu_f[pl.ds(e * G, 16)]
        xv16 = xv_f[pl.ds(e * G, 16)]
        prow = lax.rem(e, GPR) * G
        bu = jnp.full((16,), slot, jnp.int32)
        for k in range(G):
            ku = jnp.full((16,), k, jnp.int32)
            cu = jnp.full((16,), xu16[k] & 127, jnp.int32)
            cv = jnp.full((16,), xv16[k] & 127, jnp.int32)
            fu = plsc.load_gather(ubuf, [bu, ku, lane_ids, cu])
            fv = plsc.load_gather(vbuf, [bu, ku, lane_ids, cv])
            prods[prow + k] = fu * fv

    def reduce16(e):
        acc = jnp.zeros((16,), jnp.float32)
        for d in range(D):
            cols = jnp.full((16,), d, jnp.int32)
            acc = acc + plsc.load_gather(prods, [lane_ids, cols])
        out_v[pl.ds((e - (GPR - 1)) * G, 16)] = acc

    for p in range(NF):
        fire(p, p)

    def body(g, carry):
        e = g - NF

        @pl.when(g < NG)
        def _():
            for r in range(NB):
                @pl.when(lax.rem(g, NB) == r)
                def _(r=r):
                    fire(g, r)

        for r in range(NB):
            @pl.when(lax.rem(e, NB) == r)
            def _(r=r):
                drain(r)
        extract(e, lax.rem(e, NB))

        @pl.when(lax.rem(e, GPR) == GPR - 1)
        def _():
            reduce16(e)
        return carry

    lax.fori_loop(NF, NG + NF, body, 0)

    pltpu.sync_copy(out_v, out_hbm.at[pl.ds(base, BW)])


def kernel(X, user_emb, item_emb):
    xu = X[:, 0]
    xv = X[:, 1]
    out = _mf_sc(xu, xv, user_emb.T, item_emb.T)
    return out.reshape(BATCH, 1)
